# Initial kernel scaffold; baseline (speedup 1.0000x reference)
import jax
import jax.numpy as jnp
from jax.experimental import pallas as pl


def kernel(x, T, emb, W1, b1, W2, b2):
    raise NotImplementedError("write your pallas kernel here")


# speedup vs baseline: 2.5735x; 2.5735x over previous
import functools

import jax
import jax.numpy as jnp
from jax import lax
from jax.experimental import pallas as pl
from jax.experimental.pallas import tpu as pltpu
from jax.experimental.pallas import tpu_sc as plsc

B = 16384
K = 50
V = 1000000
D = 32
H = 250

NC = 2
NS = 16
NW = NC * NS
BPW = B // NW
RB = 2
RIDX = RB * K
NR = BPW // RB
NI = NR // 2


def _sc_bag_body(emb_hbm, x_hbm, w_hbm, out_hbm,
                 idx_all, w_all, rows0, rows1, out_buf,
                 sem0, sem1, sem_i, sem_w):
  wid = lax.axis_index("s") * NC + lax.axis_index("c")
  base_b = wid * BPW

  cp_i = pltpu.async_copy(x_hbm.at[pl.ds(wid * NR, NR), :], idx_all, sem_i)
  cp_w = pltpu.async_copy(w_hbm.at[pl.ds(base_b * K, BPW * K)],
                          w_all.at[pl.ds(0, BPW * K)], sem_w)
  cp_i.wait()

  def start_gather(r, buf, sem):
    pltpu.async_copy(emb_hbm.at[idx_all.at[r]], buf, sem)

  start_gather(0, rows0, sem0)
  start_gather(1, rows1, sem1)
  cp_w.wait()

  def compute_round(r, rows):
    for j in range(RB):
      b_loc = r * RB + j
      w_base = b_loc * K
      wvecs = [w_all[pl.ds(w_base + 16 * q, 16)] for q in range(4)]
      acc = [[jnp.zeros((16,), jnp.float32) for _ in range(2)]
             for _ in range(5)]
      for k in range(K):
        g = k % 5
        row = j * K + k
        wv = wvecs[k // 16][k % 16]
        acc[g][0] = acc[g][0] + wv * rows[row, pl.ds(0, 16)]
        acc[g][1] = acc[g][1] + wv * rows[row, pl.ds(16, 16)]
      lo = ((acc[0][0] + acc[1][0]) + (acc[2][0] + acc[3][0])) + acc[4][0]
      hi = ((acc[0][1] + acc[1][1]) + (acc[2][1] + acc[3][1])) + acc[4][1]
      out_buf[b_loc, pl.ds(0, 16)] = lo
      out_buf[b_loc, pl.ds(16, 16)] = hi

  def loop_body(i, _):
    r0 = i * 2
    pltpu.make_async_copy(emb_hbm.at[idx_all.at[r0]], rows0, sem0).wait()
    compute_round(r0, rows0)

    @pl.when(r0 + 2 < NR)
    def _():
      start_gather(r0 + 2, rows0, sem0)

    pltpu.make_async_copy(emb_hbm.at[idx_all.at[r0 + 1]], rows1, sem1).wait()
    compute_round(r0 + 1, rows1)

    @pl.when(r0 + 3 < NR)
    def _():
      start_gather(r0 + 3, rows1, sem1)

    return 0

  lax.fori_loop(0, NI, loop_body, 0)

  pltpu.sync_copy(out_buf, out_hbm.at[pl.ds(base_b, BPW), :])


@jax.jit
def _sc_bag(emb, x2, w_flat):
  mesh = plsc.VectorSubcoreMesh(core_axis_name="c", subcore_axis_name="s",
                                num_cores=NC, num_subcores=NS)
  f = pl.kernel(
      _sc_bag_body,
      out_type=jax.ShapeDtypeStruct((B, D), jnp.float32),
      mesh=mesh,
      compiler_params=pltpu.CompilerParams(use_tc_tiling_on_sc=False),
      scratch_types=[
          pltpu.VMEM((NR, RIDX), jnp.int32),
          pltpu.VMEM((BPW * K + 16,), jnp.float32),
          pltpu.VMEM((RIDX, D), jnp.float32),
          pltpu.VMEM((RIDX, D), jnp.float32),
          pltpu.VMEM((BPW, D), jnp.float32),
          pltpu.SemaphoreType.DMA,
          pltpu.SemaphoreType.DMA,
          pltpu.SemaphoreType.DMA,
          pltpu.SemaphoreType.DMA,
      ],
  )
  return f(emb, x2, w_flat)


def _mlp_body(mean_ref, w1_ref, b1_ref, w2_ref, b2_ref, out_ref):
  w1 = w1_ref[:] * (1.0 / K)
  h = jnp.dot(mean_ref[:], w1, preferred_element_type=jnp.float32)
  h = jnp.maximum(h + b1_ref[:], 0.0)
  out_ref[:] = jnp.dot(h, w2_ref[:], preferred_element_type=jnp.float32) \
      + b2_ref[:]


@jax.jit
def _mlp(mean, W1, b1, W2, b2):
  M = 2048
  grid = (B // M,)
  return pl.pallas_call(
      _mlp_body,
      grid=grid,
      in_specs=[
          pl.BlockSpec((M, D), lambda i: (i, 0)),
          pl.BlockSpec((D, H), lambda i: (0, 0)),
          pl.BlockSpec((1, H), lambda i: (0, 0)),
          pl.BlockSpec((H, 2), lambda i: (0, 0)),
          pl.BlockSpec((1, 2), lambda i: (0, 0)),
      ],
      out_specs=pl.BlockSpec((M, 2), lambda i: (i, 0)),
      out_shape=jax.ShapeDtypeStruct((B, 2), jnp.float32),
  )(mean, W1, b1, W2, b2)


def kernel(x, T, emb, W1, b1, W2, b2):
  x2 = x.astype(jnp.int32).reshape(B // RB, RIDX)
  w_flat = T.reshape(B * K)
  mean_sum = _sc_bag(emb, x2, w_flat)
  return _mlp(mean_sum, W1, b1.reshape(1, H), W2, b2.reshape(1, 2))
